# baseline (speedup 1.0000x reference)
import functools
import jax
import jax.numpy as jnp
from jax import lax
from jax.experimental import pallas as pl
from jax.experimental.pallas import tpu as pltpu
from jax.experimental.pallas import tpu_sc as plsc

_K = 1024
_D = 64
_R = 4096
_CC = 0.25
_NW = 32
_CHUNK = 128


def _vq_body(x_ref, cbt2_ref, x2_ref, e2_ref, iota_ref, idx_ref, loss_ref):
    i = pl.program_id(0)
    xe2 = jax.lax.dot_general(x_ref[...], cbt2_ref[...],
                              (((1,), (0,)), ((), ())),
                              preferred_element_type=jnp.float32)
    d = (x2_ref[...] + e2_ref[...]) - xe2
    m = jnp.min(d, axis=1, keepdims=True)
    iota = iota_ref[...]
    idx_ref[...] = jnp.min(jnp.where(d == m, iota, float(_K)), axis=1,
                           keepdims=True)
    part = jnp.sum(m).reshape(1, 1)

    @pl.when(i == 0)
    def _init():
        loss_ref[...] = part

    @pl.when(i != 0)
    def _acc():
        loss_ref[...] += part


def _sc_gather(cb_hbm, idx_hbm, out_hbm, idx_v, buf0, buf1, sem0, sem1):
    wid = lax.axis_index("s") * 2 + lax.axis_index("c")
    b_per_w = (65536) // _NW
    base = wid * b_per_w
    pltpu.sync_copy(idx_hbm.at[pl.ds(base, b_per_w)], idx_v)
    bufs = (buf0, buf1)
    sems = (sem0, sem1)
    n_chunks = b_per_w // _CHUNK
    pend = {}
    for j in range(2):
        pend[j] = pltpu.async_copy(
            cb_hbm.at[idx_v.at[pl.ds(j * _CHUNK, _CHUNK)]], bufs[j], sems[j])
    for j in range(n_chunks):
        pend[j].wait()
        pltpu.sync_copy(bufs[j % 2], out_hbm.at[pl.ds(base + j * _CHUNK, _CHUNK)])
        nxt = j + 2
        if nxt < n_chunks:
            pend[nxt] = pltpu.async_copy(
                cb_hbm.at[idx_v.at[pl.ds(nxt * _CHUNK, _CHUNK)]],
                bufs[j % 2], sems[j % 2])


def kernel(z, codebook):
    n = z.shape[0] * z.shape[1]
    flat = z.reshape(n, _D)
    cbt2 = 2.0 * codebook.T
    x2 = jnp.sum(flat ** 2, axis=1, keepdims=True)
    e2 = jnp.sum(codebook ** 2, axis=1)[None, :]
    grid = n // _R
    idxf, loss_sum = pl.pallas_call(
        _vq_body,
        grid=(grid,),
        in_specs=[
            pl.BlockSpec((_R, _D), lambda i: (i, 0)),
            pl.BlockSpec((_D, _K), lambda i: (0, 0)),
            pl.BlockSpec((_R, 1), lambda i: (i, 0)),
            pl.BlockSpec((1, _K), lambda i: (0, 0)),
            pl.BlockSpec((1, _K), lambda i: (0, 0)),
        ],
        out_specs=[
            pl.BlockSpec((_R, 1), lambda i: (i, 0)),
            pl.BlockSpec((1, 1), lambda i: (0, 0)),
        ],
        out_shape=[
            jax.ShapeDtypeStruct((n, 1), jnp.float32),
            jax.ShapeDtypeStruct((1, 1), jnp.float32),
        ],
    )(flat, cbt2, x2, e2, jnp.arange(_K, dtype=jnp.float32)[None, :])

    idx = idxf[:, 0].astype(jnp.int32)
    b_per_w = n // _NW
    mesh = plsc.VectorSubcoreMesh(core_axis_name="c", subcore_axis_name="s")
    gather = functools.partial(
        pl.kernel, mesh=mesh,
        compiler_params=pltpu.CompilerParams(use_tc_tiling_on_sc=False),
        out_type=jax.ShapeDtypeStruct((n, _D), jnp.float32),
        scratch_types=[
            pltpu.VMEM((b_per_w,), jnp.int32),
            pltpu.VMEM((_CHUNK, _D), jnp.float32),
            pltpu.VMEM((_CHUNK, _D), jnp.float32),
            pltpu.SemaphoreType.DMA,
            pltpu.SemaphoreType.DMA,
        ],
    )(_sc_gather)
    out = gather(codebook, idx)
    m = loss_sum[0, 0] / (n * _D)
    loss = m + _CC * m
    return out.reshape(z.shape), loss


# speedup vs baseline: 1.1048x; 1.1048x over previous
import jax
import jax.numpy as jnp
from jax.experimental import pallas as pl

_K = 1024
_D = 64
_R = 4096
_CC = 0.25


def _vq_body(x_ref, cbt2_ref, cb_ref, x2_ref, e2_ref, iota_ref, out_ref,
             loss_ref):
    i = pl.program_id(0)
    xe2 = jax.lax.dot_general(x_ref[...], cbt2_ref[...],
                              (((1,), (0,)), ((), ())),
                              preferred_element_type=jnp.float32)
    d = (x2_ref[...] + e2_ref[...]) - xe2
    m = jnp.min(d, axis=1, keepdims=True)
    iota = iota_ref[...]
    idx = jnp.min(jnp.where(d == m, iota, float(_K)), axis=1)
    onehot = (iota == idx[:, None]).astype(jnp.float32)
    q = jax.lax.dot_general(onehot, cb_ref[...], (((1,), (0,)), ((), ())),
                            preferred_element_type=jnp.float32)
    out_ref[...] = q
    part = jnp.sum(m).reshape(1, 1)

    @pl.when(i == 0)
    def _init():
        loss_ref[...] = part

    @pl.when(i != 0)
    def _acc():
        loss_ref[...] += part


def kernel(z, codebook):
    n = z.shape[0] * z.shape[1]
    flat = z.reshape(n, _D)
    cbt2 = 2.0 * codebook.T
    x2 = jnp.sum(flat ** 2, axis=1, keepdims=True)
    e2 = jnp.sum(codebook ** 2, axis=1)[None, :]
    grid = n // _R
    out, loss_sum = pl.pallas_call(
        _vq_body,
        grid=(grid,),
        in_specs=[
            pl.BlockSpec((_R, _D), lambda i: (i, 0)),
            pl.BlockSpec((_D, _K), lambda i: (0, 0)),
            pl.BlockSpec((_K, _D), lambda i: (0, 0)),
            pl.BlockSpec((_R, 1), lambda i: (i, 0)),
            pl.BlockSpec((1, _K), lambda i: (0, 0)),
            pl.BlockSpec((1, _K), lambda i: (0, 0)),
        ],
        out_specs=[
            pl.BlockSpec((_R, _D), lambda i: (i, 0)),
            pl.BlockSpec((1, 1), lambda i: (0, 0)),
        ],
        out_shape=[
            jax.ShapeDtypeStruct((n, _D), jnp.float32),
            jax.ShapeDtypeStruct((1, 1), jnp.float32),
        ],
    )(flat, cbt2, codebook, x2, e2, jnp.arange(_K, dtype=jnp.float32)[None, :])
    m = loss_sum[0, 0] / (n * _D)
    loss = m + _CC * m
    return out.reshape(z.shape), loss
